# baseline (speedup 1.0000x reference)
import functools

import jax
import jax.numpy as jnp
from jax import lax
from jax.experimental import pallas as pl
from jax.experimental.pallas import tpu as pltpu
from jax.experimental.pallas import tpu_sc as plsc

_CH = 128


def _sc_info():
    info = plsc.get_sparse_core_info()
    return info.num_cores, info.num_subcores


def _row_split(n, ns):
    per_tile = ((n + ns - 1) // ns + 7) // 8 * 8
    if per_tile * ns <= n:
        per_tile += 8
    return per_tile, per_tile * ns


def _zero_fill(ref, nrows, ncols):
    per_row = ncols // 16

    def body(i, _):
        r = i // per_row
        cpos = i % per_row
        ref[r, pl.ds(cpos * 16, 16)] = jnp.zeros((16,), jnp.float32)
        return 0

    lax.fori_loop(0, nrows * per_row, body, 0)


def _fill_flat(ref, nelems, value):
    def body(i, _):
        ref[pl.ds(i * 16, 16)] = jnp.full((16,), value, jnp.float32)
        return 0

    lax.fori_loop(0, nelems // 16, body, 0)


def _sc_degrees(src2d, dst2d, n):
    nck = src2d.shape[0]
    nc, ns = _sc_info()
    nw = nc * ns
    cpt = nck // nw
    grp = 4
    per_tile, npad = _row_split(n, ns)
    mesh = plsc.VectorSubcoreMesh(core_axis_name="c", subcore_axis_name="s")

    @functools.partial(
        pl.kernel,
        out_type=(
            jax.ShapeDtypeStruct((nc * npad,), jnp.float32),
            jax.ShapeDtypeStruct((nc * npad,), jnp.float32),
        ),
        mesh=mesh,
        scratch_types=[
            pltpu.VMEM((cpt, _CH), jnp.int32),
            pltpu.VMEM((cpt, _CH), jnp.int32),
            pltpu.VMEM((_CH,), jnp.float32),
            pltpu.VMEM((per_tile,), jnp.float32),
            pltpu.VMEM_SHARED((npad,), jnp.float32),
            pltpu.VMEM_SHARED((npad,), jnp.float32),
            pltpu.SemaphoreType.DMA,
        ],
        compiler_params=pltpu.CompilerParams(use_tc_tiling_on_sc=False),
    )
    def deg_kernel(src_hbm, dst_hbm, out_s, out_d, sidx, didx, ones, buf,
                   deg_s, deg_d, sem):
        c = lax.axis_index("c")
        s = lax.axis_index("s")
        wid = s * nc + c
        base = s * per_tile

        pltpu.sync_copy(src_hbm.at[pl.ds(wid * cpt, cpt)], sidx)
        pltpu.sync_copy(dst_hbm.at[pl.ds(wid * cpt, cpt)], didx)
        _fill_flat(ones, _CH, 1.0)
        _fill_flat(buf, per_tile, 0.0)
        pltpu.sync_copy(buf, deg_s.at[pl.ds(base, per_tile)])
        pltpu.sync_copy(buf, deg_d.at[pl.ds(base, per_tile)])
        plsc.subcore_barrier()

        def body(i, _):
            for j in range(grp):
                t = i * grp + j
                pltpu.async_copy(ones, deg_s.at[sidx.at[t]], sem, add=True)
                pltpu.async_copy(ones, deg_d.at[didx.at[t]], sem, add=True)
            for j in range(2 * grp):
                pltpu.make_async_copy(ones, deg_s.at[sidx.at[i * grp]],
                                      sem).wait()
            return 0

        lax.fori_loop(0, cpt // grp, body, 0)
        plsc.subcore_barrier()

        obase = c * npad + base
        pltpu.sync_copy(deg_s.at[pl.ds(base, per_tile)], buf)
        pltpu.sync_copy(buf, out_s.at[pl.ds(obase, per_tile)])
        pltpu.sync_copy(deg_d.at[pl.ds(base, per_tile)], buf)
        pltpu.sync_copy(buf, out_d.at[pl.ds(obase, per_tile)])

    out_s, out_d = deg_kernel(src2d, dst2d)
    return out_s.reshape(nc, npad), out_d.reshape(nc, npad)


def _sc_scatter(src2d, dst2d, table, n, cpt0, cpt1):
    h = table.shape[1]
    nc, ns = _sc_info()
    nbuf = 4
    per_tile, npad = _row_split(n, ns)
    full = per_tile // _CH
    tail = per_tile - full * _CH
    mesh = plsc.VectorSubcoreMesh(core_axis_name="c", subcore_axis_name="s")

    @functools.partial(
        pl.kernel,
        out_type=jax.ShapeDtypeStruct((nc, npad, h), jnp.float32),
        mesh=mesh,
        scratch_types=[
            pltpu.VMEM((cpt0, _CH), jnp.int32),
            pltpu.VMEM((cpt0, _CH), jnp.int32),
        ] + [pltpu.VMEM((_CH, h), jnp.float32) for _ in range(nbuf)] + [
            pltpu.VMEM_SHARED((npad, h), jnp.float32),
        ] + [pltpu.SemaphoreType.DMA for _ in range(nbuf)],
        compiler_params=pltpu.CompilerParams(use_tc_tiling_on_sc=False),
    )
    def scatter_kernel(src_hbm, dst_hbm, table_hbm, out_hbm, sidx, didx,
                       *bufs_agg_sems):
        rows = bufs_agg_sems[:nbuf]
        agg = bufs_agg_sems[nbuf]
        sems = bufs_agg_sems[nbuf + 1:]
        c = lax.axis_index("c")
        s = lax.axis_index("s")
        base = s * per_tile
        my_cpt = cpt0 + (cpt1 - cpt0) * c
        start = c * ns * cpt0 + s * my_cpt

        pltpu.sync_copy(src_hbm.at[pl.ds(start, cpt0)], sidx)
        pltpu.sync_copy(dst_hbm.at[pl.ds(start, cpt0)], didx)

        _zero_fill(rows[0], _CH, h)
        for k in range(full):
            pltpu.sync_copy(rows[0], agg.at[pl.ds(base + k * _CH, _CH)])
        if tail:
            pltpu.sync_copy(rows[0].at[pl.ds(0, tail)],
                            agg.at[pl.ds(base + full * _CH, tail)])
        plsc.subcore_barrier()

        for j in range(nbuf - 1):
            pltpu.async_copy(table_hbm.at[sidx.at[j]], rows[j], sems[j])

        def body(i, _):
            for j in range(nbuf):
                t = i * nbuf + j

                @pl.when(t < my_cpt)
                def _():
                    pltpu.make_async_copy(table_hbm.at[sidx.at[t]], rows[j],
                                          sems[j]).wait()
                    pltpu.sync_copy(rows[j], agg.at[didx.at[t]], add=True)

                ta = t + nbuf - 1
                ja = (j + nbuf - 1) % nbuf

                @pl.when(ta < my_cpt)
                def _():
                    pltpu.async_copy(table_hbm.at[sidx.at[ta]], rows[ja],
                                     sems[ja])

            return 0

        lax.fori_loop(0, cpt0 // nbuf, body, 0)
        plsc.subcore_barrier()

        for k in range(full + 1):
            sz = _CH if k < full else tail
            if sz:
                ro = base + k * _CH
                pltpu.sync_copy(agg.at[pl.ds(ro, sz)],
                                rows[0].at[pl.ds(0, sz)])
                pltpu.sync_copy(rows[0].at[pl.ds(0, sz)],
                                out_hbm.at[c, pl.ds(ro, sz)])

    return scatter_kernel(src2d, dst2d, table)


def _norm_from(deg_ref):
    d = jnp.sum(deg_ref[...], axis=1, keepdims=True)
    return lax.rsqrt(jnp.maximum(d, 1.0))


def _tc_table0(x, w0, deg_s, n, blk):
    d_in = x.shape[1]
    h = w0.shape[1]

    def body(x_ref, w_ref, ds_ref, o_ref):
        sn = _norm_from(ds_ref)
        o_ref[...] = (
            jnp.dot(x_ref[...], w_ref[...], preferred_element_type=jnp.float32)
            * sn)

    return pl.pallas_call(
        body,
        grid=(n // blk,),
        in_specs=[
            pl.BlockSpec((blk, d_in), lambda i: (i, 0)),
            pl.BlockSpec((d_in, h), lambda i: (0, 0)),
            pl.BlockSpec((blk, deg_s.shape[1]), lambda i: (i, 0)),
        ],
        out_specs=pl.BlockSpec((blk, h), lambda i: (i, 0)),
        out_shape=jax.ShapeDtypeStruct((n, h), jnp.float32),
    )(x, w0, deg_s)


def _tc_table1(agg_p, deg_s, deg_d, b0, n, blk):
    h = agg_p.shape[2]

    def body(p_ref, ds_ref, dd_ref, b_ref, o_ref):
        a = jnp.sum(p_ref[...], axis=0)
        hid = jnp.maximum(a * _norm_from(dd_ref) + b_ref[...], 0.0)
        o_ref[...] = hid * _norm_from(ds_ref)

    return pl.pallas_call(
        body,
        grid=(n // blk,),
        in_specs=[
            pl.BlockSpec((agg_p.shape[0], blk, h), lambda i: (0, i, 0)),
            pl.BlockSpec((blk, deg_s.shape[1]), lambda i: (i, 0)),
            pl.BlockSpec((blk, deg_d.shape[1]), lambda i: (i, 0)),
            pl.BlockSpec((1, h), lambda i: (0, 0)),
        ],
        out_specs=pl.BlockSpec((blk, h), lambda i: (i, 0)),
        out_shape=jax.ShapeDtypeStruct((n, h), jnp.float32),
    )(agg_p, deg_s, deg_d, b0)


def _tc_final(agg_p, deg_d, w1, b1, w2, b2, noise, n, blk):
    h = agg_p.shape[2]
    ho = w1.shape[1]

    def body(p_ref, dd_ref, w1_ref, b1_ref, w2_ref, b2_ref, nz_ref, o_ref):
        m = jnp.sum(p_ref[...], axis=0) * _norm_from(dd_ref)
        mean = jnp.dot(m, w1_ref[...],
                       preferred_element_type=jnp.float32) + b1_ref[...]
        log_std = jnp.dot(m, w2_ref[...],
                          preferred_element_type=jnp.float32) + b2_ref[...]
        o_ref[...] = mean + nz_ref[...] * jnp.exp(log_std)

    return pl.pallas_call(
        body,
        grid=(n // blk,),
        in_specs=[
            pl.BlockSpec((agg_p.shape[0], blk, h), lambda i: (0, i, 0)),
            pl.BlockSpec((blk, deg_d.shape[1]), lambda i: (i, 0)),
            pl.BlockSpec((h, ho), lambda i: (0, 0)),
            pl.BlockSpec((1, ho), lambda i: (0, 0)),
            pl.BlockSpec((h, ho), lambda i: (0, 0)),
            pl.BlockSpec((1, ho), lambda i: (0, 0)),
            pl.BlockSpec((blk, ho), lambda i: (i, 0)),
        ],
        out_specs=pl.BlockSpec((blk, ho), lambda i: (i, 0)),
        out_shape=jax.ShapeDtypeStruct((n, ho), jnp.float32),
    )(agg_p, deg_d, w1, b1, w2, b2, noise)


def kernel(in_feat, edge_index, W0, b0, W1, b1, W2, b2, noise):
    n = in_feat.shape[0]
    e = edge_index.shape[1]
    blk = 1000
    src = edge_index[0].astype(jnp.int32)
    dst = edge_index[1].astype(jnp.int32)

    nc, ns = _sc_info()
    nchunk = -(-e // _CH)
    cpt_d = -(-nchunk // (nc * ns))
    cpt_d = (cpt_d + 3) // 4 * 4
    nproc_d = nc * ns * cpt_d
    u = -(-nchunk // (ns * 15))
    u = (u + 3) // 4 * 4
    cpt0, cpt1 = 14 * u, u
    nproc_s = ns * (cpt0 + cpt1)
    slack = cpt0 - cpt1
    epad = max(nproc_d, nproc_s + slack) * _CH
    pad = epad - e
    _, npad_rows = _row_split(n, ns)
    sent = n + jnp.arange(pad, dtype=jnp.int32) % (npad_rows - n)
    src_g = jnp.concatenate([src, jnp.zeros((pad,), jnp.int32)])
    src_s = jnp.concatenate([src, sent])
    dst_s = jnp.concatenate([dst, sent])
    src_g2 = src_g.reshape(-1, _CH)
    src_s2 = src_s.reshape(-1, _CH)
    dst_s2 = dst_s.reshape(-1, _CH)

    deg_s_p, deg_d_p = _sc_degrees(src_s2[:nproc_d], dst_s2[:nproc_d], n)
    deg_s = deg_s_p.T
    deg_d = deg_d_p.T

    table0 = _tc_table0(in_feat, W0, deg_s, n, blk)
    agg0_p = _sc_scatter(src_g2[:nproc_s + slack], dst_s2[:nproc_s + slack],
                         table0, n, cpt0, cpt1)
    table1 = _tc_table1(agg0_p, deg_s, deg_d, b0[None, :], n, blk)
    agg1_p = _sc_scatter(src_g2[:nproc_s + slack], dst_s2[:nproc_s + slack],
                         table1, n, cpt0, cpt1)
    return _tc_final(agg1_p, deg_d, W1, b1[None, :], W2, b2[None, :], noise,
                     n, blk)


# speedup vs baseline: 4.4612x; 4.4612x over previous
import functools

import jax
import jax.numpy as jnp
from jax import lax
from jax.experimental import pallas as pl
from jax.experimental.pallas import tpu as pltpu
from jax.experimental.pallas import tpu_sc as plsc

_CH = 128


def _sc_info():
    info = plsc.get_sparse_core_info()
    return info.num_cores, info.num_subcores


def _row_split(n, ns):
    per_tile = ((n + ns - 1) // ns + 7) // 8 * 8
    if per_tile * ns <= n:
        per_tile += 8
    return per_tile, per_tile * ns


def _zero_fill(ref, nrows, ncols):
    per_row = ncols // 16

    def body(i, _):
        r = i // per_row
        cpos = i % per_row
        ref[r, pl.ds(cpos * 16, 16)] = jnp.zeros((16,), jnp.float32)
        return 0

    lax.fori_loop(0, nrows * per_row, body, 0)


def _fill_flat(ref, nelems, value):
    def body(i, _):
        ref[pl.ds(i * 16, 16)] = jnp.full((16,), value, jnp.float32)
        return 0

    lax.fori_loop(0, nelems // 16, body, 0)


def _sc_degrees(src2d, dst2d, n):
    nck = src2d.shape[0]
    nc, ns = _sc_info()
    nw = nc * ns
    cpt = nck // nw
    grp = 4
    per_tile, npad = _row_split(n, ns)
    mesh = plsc.VectorSubcoreMesh(core_axis_name="c", subcore_axis_name="s")

    @functools.partial(
        pl.kernel,
        out_type=(
            jax.ShapeDtypeStruct((nc * npad,), jnp.float32),
            jax.ShapeDtypeStruct((nc * npad,), jnp.float32),
        ),
        mesh=mesh,
        scratch_types=[
            pltpu.VMEM((cpt, _CH), jnp.int32),
            pltpu.VMEM((cpt, _CH), jnp.int32),
            pltpu.VMEM((_CH,), jnp.float32),
            pltpu.VMEM((per_tile,), jnp.float32),
            pltpu.VMEM_SHARED((npad,), jnp.float32),
            pltpu.VMEM_SHARED((npad,), jnp.float32),
            pltpu.SemaphoreType.DMA,
        ],
        compiler_params=pltpu.CompilerParams(use_tc_tiling_on_sc=False),
    )
    def deg_kernel(src_hbm, dst_hbm, out_s, out_d, sidx, didx, ones, buf,
                   deg_s, deg_d, sem):
        c = lax.axis_index("c")
        s = lax.axis_index("s")
        wid = s * nc + c
        base = s * per_tile

        pltpu.sync_copy(src_hbm.at[pl.ds(wid * cpt, cpt)], sidx)
        pltpu.sync_copy(dst_hbm.at[pl.ds(wid * cpt, cpt)], didx)
        _fill_flat(ones, _CH, 1.0)
        _fill_flat(buf, per_tile, 0.0)
        pltpu.sync_copy(buf, deg_s.at[pl.ds(base, per_tile)])
        pltpu.sync_copy(buf, deg_d.at[pl.ds(base, per_tile)])
        plsc.subcore_barrier()

        def body(i, _):
            for j in range(grp):
                t = i * grp + j
                pltpu.async_copy(ones, deg_s.at[sidx.at[t]], sem, add=True)
                pltpu.async_copy(ones, deg_d.at[didx.at[t]], sem, add=True)
            for j in range(2 * grp):
                pltpu.make_async_copy(ones, deg_s.at[sidx.at[i * grp]],
                                      sem).wait()
            return 0

        lax.fori_loop(0, cpt // grp, body, 0)
        plsc.subcore_barrier()

        obase = c * npad + base
        pltpu.sync_copy(deg_s.at[pl.ds(base, per_tile)], buf)
        pltpu.sync_copy(buf, out_s.at[pl.ds(obase, per_tile)])
        pltpu.sync_copy(deg_d.at[pl.ds(base, per_tile)], buf)
        pltpu.sync_copy(buf, out_d.at[pl.ds(obase, per_tile)])

    out_s, out_d = deg_kernel(src2d, dst2d)
    return out_s.reshape(nc, npad), out_d.reshape(nc, npad)


def _sc_scatter(src2d, dst2d, table, n, cpt0, cpt1):
    h = table.shape[1]
    nc, ns = _sc_info()
    nbuf = 4
    per_tile, npad = _row_split(n, ns)
    full = per_tile // _CH
    tail = per_tile - full * _CH
    mesh = plsc.VectorSubcoreMesh(core_axis_name="c", subcore_axis_name="s")

    @functools.partial(
        pl.kernel,
        out_type=jax.ShapeDtypeStruct((nc, npad, h), jnp.float32),
        mesh=mesh,
        scratch_types=[
            pltpu.VMEM((cpt0, _CH), jnp.int32),
            pltpu.VMEM((cpt0, _CH), jnp.int32),
        ] + [pltpu.VMEM((_CH, h), jnp.float32) for _ in range(nbuf)] + [
            pltpu.VMEM_SHARED((npad, h), jnp.float32),
        ] + [pltpu.SemaphoreType.DMA for _ in range(nbuf)],
        compiler_params=pltpu.CompilerParams(use_tc_tiling_on_sc=False),
    )
    def scatter_kernel(src_hbm, dst_hbm, table_hbm, out_hbm, sidx, didx,
                       *bufs_agg_sems):
        rows = bufs_agg_sems[:nbuf]
        agg = bufs_agg_sems[nbuf]
        sems = bufs_agg_sems[nbuf + 1:]
        c = lax.axis_index("c")
        s = lax.axis_index("s")
        base = s * per_tile
        my_cpt = cpt0 + (cpt1 - cpt0) * c
        start = c * ns * cpt0 + s * my_cpt

        pltpu.sync_copy(src_hbm.at[pl.ds(start, cpt0)], sidx)
        pltpu.sync_copy(dst_hbm.at[pl.ds(start, cpt0)], didx)

        _zero_fill(rows[0], _CH, h)
        for k in range(full):
            pltpu.sync_copy(rows[0], agg.at[pl.ds(base + k * _CH, _CH)])
        if tail:
            pltpu.sync_copy(rows[0].at[pl.ds(0, tail)],
                            agg.at[pl.ds(base + full * _CH, tail)])
        plsc.subcore_barrier()

        for j in range(nbuf - 1):
            pltpu.async_copy(table_hbm.at[sidx.at[j]], rows[j], sems[j])

        def body(i, _):
            for j in range(nbuf):
                t = i * nbuf + j

                @pl.when(t < my_cpt)
                def _():
                    pltpu.make_async_copy(table_hbm.at[sidx.at[t]], rows[j],
                                          sems[j]).wait()
                    pltpu.sync_copy(rows[j], agg.at[didx.at[t]], add=True)

                ta = t + nbuf - 1
                ja = (j + nbuf - 1) % nbuf

                @pl.when(ta < my_cpt)
                def _():
                    pltpu.async_copy(table_hbm.at[sidx.at[ta]], rows[ja],
                                     sems[ja])

            return 0

        lax.fori_loop(0, cpt0 // nbuf, body, 0)
        plsc.subcore_barrier()

        for k in range(full + 1):
            sz = _CH if k < full else tail
            if sz:
                ro = base + k * _CH
                pltpu.sync_copy(agg.at[pl.ds(ro, sz)],
                                rows[0].at[pl.ds(0, sz)])
                pltpu.sync_copy(rows[0].at[pl.ds(0, sz)],
                                out_hbm.at[c, pl.ds(ro, sz)])

    return scatter_kernel(src2d, dst2d, table)


def _norm_from(deg_ref):
    d = jnp.sum(deg_ref[...], axis=1, keepdims=True)
    return lax.rsqrt(jnp.maximum(d, 1.0))


def _tc_table0(x, w0, deg_s, n, blk):
    d_in = x.shape[1]
    h = w0.shape[1]

    def body(x_ref, w_ref, ds_ref, o_ref):
        sn = _norm_from(ds_ref)
        o_ref[...] = (
            jnp.dot(x_ref[...], w_ref[...], preferred_element_type=jnp.float32)
            * sn)

    return pl.pallas_call(
        body,
        grid=(n // blk,),
        in_specs=[
            pl.BlockSpec((blk, d_in), lambda i: (i, 0)),
            pl.BlockSpec((d_in, h), lambda i: (0, 0)),
            pl.BlockSpec((blk, deg_s.shape[1]), lambda i: (i, 0)),
        ],
        out_specs=pl.BlockSpec((blk, h), lambda i: (i, 0)),
        out_shape=jax.ShapeDtypeStruct((n, h), jnp.float32),
    )(x, w0, deg_s)


def _tc_table1(agg_p, deg_s, deg_d, b0, n, blk):
    h = agg_p.shape[2]

    def body(p_ref, ds_ref, dd_ref, b_ref, o_ref):
        a = jnp.sum(p_ref[...], axis=0)
        hid = jnp.maximum(a * _norm_from(dd_ref) + b_ref[...], 0.0)
        o_ref[...] = hid * _norm_from(ds_ref)

    return pl.pallas_call(
        body,
        grid=(n // blk,),
        in_specs=[
            pl.BlockSpec((agg_p.shape[0], blk, h), lambda i: (0, i, 0)),
            pl.BlockSpec((blk, deg_s.shape[1]), lambda i: (i, 0)),
            pl.BlockSpec((blk, deg_d.shape[1]), lambda i: (i, 0)),
            pl.BlockSpec((1, h), lambda i: (0, 0)),
        ],
        out_specs=pl.BlockSpec((blk, h), lambda i: (i, 0)),
        out_shape=jax.ShapeDtypeStruct((n, h), jnp.float32),
    )(agg_p, deg_s, deg_d, b0)


def _tc_final(agg_p, deg_d, w1, b1, w2, b2, noise, n, blk):
    h = agg_p.shape[2]
    ho = w1.shape[1]

    def body(p_ref, dd_ref, w1_ref, b1_ref, w2_ref, b2_ref, nz_ref, o_ref):
        m = jnp.sum(p_ref[...], axis=0) * _norm_from(dd_ref)
        mean = jnp.dot(m, w1_ref[...],
                       preferred_element_type=jnp.float32) + b1_ref[...]
        log_std = jnp.dot(m, w2_ref[...],
                          preferred_element_type=jnp.float32) + b2_ref[...]
        o_ref[...] = mean + nz_ref[...] * jnp.exp(log_std)

    return pl.pallas_call(
        body,
        grid=(n // blk,),
        in_specs=[
            pl.BlockSpec((agg_p.shape[0], blk, h), lambda i: (0, i, 0)),
            pl.BlockSpec((blk, deg_d.shape[1]), lambda i: (i, 0)),
            pl.BlockSpec((h, ho), lambda i: (0, 0)),
            pl.BlockSpec((1, ho), lambda i: (0, 0)),
            pl.BlockSpec((h, ho), lambda i: (0, 0)),
            pl.BlockSpec((1, ho), lambda i: (0, 0)),
            pl.BlockSpec((blk, ho), lambda i: (i, 0)),
        ],
        out_specs=pl.BlockSpec((blk, ho), lambda i: (i, 0)),
        out_shape=jax.ShapeDtypeStruct((n, ho), jnp.float32),
    )(agg_p, deg_d, w1, b1, w2, b2, noise)


def kernel(in_feat, edge_index, W0, b0, W1, b1, W2, b2, noise):
    n = in_feat.shape[0]
    e = edge_index.shape[1]
    blk = 1000
    src = edge_index[0].astype(jnp.int32)
    dst = edge_index[1].astype(jnp.int32)

    nc, ns = _sc_info()
    nchunk = -(-e // _CH)
    cpt_d = -(-nchunk // (nc * ns))
    cpt_d = (cpt_d + 3) // 4 * 4
    nproc_d = nc * ns * cpt_d
    u = -(-nchunk // (ns * 10))
    u = (u + 3) // 4 * 4
    cpt0, cpt1 = 9 * u, u
    nproc_s = ns * (cpt0 + cpt1)
    slack = cpt0 - cpt1
    epad = max(nproc_d, nproc_s + slack) * _CH
    pad = epad - e
    _, npad_rows = _row_split(n, ns)
    sent = n + jnp.arange(pad, dtype=jnp.int32) % (npad_rows - n)
    src_g = jnp.concatenate([src, jnp.zeros((pad,), jnp.int32)])
    src_s = jnp.concatenate([src, sent])
    dst_s = jnp.concatenate([dst, sent])
    src_g2 = src_g.reshape(-1, _CH)
    src_s2 = src_s.reshape(-1, _CH)
    dst_s2 = dst_s.reshape(-1, _CH)

    deg_s_p, deg_d_p = _sc_degrees(src_s2[:nproc_d], dst_s2[:nproc_d], n)
    deg_s = deg_s_p.T
    deg_d = deg_d_p.T

    table0 = _tc_table0(in_feat, W0, deg_s, n, blk)
    agg0_p = _sc_scatter(src_g2[:nproc_s + slack], dst_s2[:nproc_s + slack],
                         table0, n, cpt0, cpt1)
    table1 = _tc_table1(agg0_p, deg_s, deg_d, b0[None, :], n, blk)
    agg1_p = _sc_scatter(src_g2[:nproc_s + slack], dst_s2[:nproc_s + slack],
                         table1, n, cpt0, cpt1)
    return _tc_final(agg1_p, deg_d, W1, b1[None, :], W2, b2[None, :], noise,
                     n, blk)
